# speedup vs baseline: 2.8628x; 2.8628x over previous
import functools

import jax
import jax.numpy as jnp
from jax import lax
from jax.experimental import pallas as pl
from jax.experimental.pallas import tpu as pltpu
from jax.experimental.pallas import tpu_sc as plsc

N_NODES = 10000
N_EDGES = 320000
D_FEAT = 128

NUM_WORKERS = 32
E_PER_W = N_EDGES // NUM_WORKERS
CHUNK = 80
CHUNKS_PER_W = E_PER_W // CHUNK


def _gather_kernel(x_hbm, src_hbm, out_hbm, idx_v, rows_v, sem):
    cid = lax.axis_index("c")
    sid = lax.axis_index("s")
    wid = sid * 2 + cid
    base = wid * E_PER_W

    def body(i, carry):
        off = pl.multiple_of(base + i * CHUNK, 8)
        pltpu.sync_copy(src_hbm.at[pl.ds(off, CHUNK)], idx_v)
        pltpu.async_copy(x_hbm.at[idx_v], rows_v, sem).wait()
        pltpu.sync_copy(rows_v, out_hbm.at[pl.ds(off, CHUNK)])
        return carry

    lax.fori_loop(0, CHUNKS_PER_W, body, 0)


def kernel(x, edge_index):
    src = edge_index[0]
    mesh = plsc.VectorSubcoreMesh(core_axis_name="c", subcore_axis_name="s")
    run = functools.partial(
        pl.kernel,
        out_type=jax.ShapeDtypeStruct((N_EDGES, D_FEAT), jnp.float32),
        mesh=mesh,
        scratch_types=[
            pltpu.VMEM((CHUNK,), jnp.int32),
            pltpu.VMEM((CHUNK, D_FEAT), jnp.float32),
            pltpu.SemaphoreType.DMA,
        ],
    )(_gather_kernel)
    return run(x, src)


# speedup vs baseline: 5.7208x; 1.9984x over previous
import functools

import jax
import jax.numpy as jnp
from jax import lax
from jax.experimental import pallas as pl
from jax.experimental.pallas import tpu as pltpu
from jax.experimental.pallas import tpu_sc as plsc

N_NODES = 10000
N_EDGES = 320000
D_FEAT = 128

NUM_WORKERS = 32
E_PER_W = N_EDGES // NUM_WORKERS
SUPER = 400
G = 80
NG = SUPER // G
NSUP = E_PER_W // SUPER


def _gather_kernel(x_hbm, src_hbm, out_hbm,
                   idx0, idx1, rows0, rows1,
                   gsem0, gsem1, ssem0, ssem1):
    cid = lax.axis_index("c")
    sid = lax.axis_index("s")
    wid = sid * 2 + cid
    base = wid * E_PER_W

    def issue(i, idxb, rowsb, gsem):
        off = pl.multiple_of(base + i * SUPER, 8)
        pltpu.sync_copy(src_hbm.at[pl.ds(off, SUPER)], idxb)
        for j in range(NG):
            pltpu.async_copy(
                x_hbm.at[idxb.at[pl.ds(j * G, G)]],
                rowsb.at[pl.ds(j * G, G)],
                gsem,
            )

    def drain_gathers(rowsb, gsem):
        for j in range(NG):
            pltpu.make_async_copy(
                x_hbm.at[pl.ds(0, G)], rowsb.at[pl.ds(j * G, G)], gsem
            ).wait()

    def wait_store(rowsb, ssem):
        pltpu.make_async_copy(rowsb, out_hbm.at[pl.ds(0, SUPER)], ssem).wait()

    def step(i, idx_c, rows_c, gsem_c, ssem_c, idx_n, rows_n, gsem_n, ssem_n):
        @pl.when(i + 1 < NSUP)
        def _():
            @pl.when(i >= 1)
            def _():
                wait_store(rows_n, ssem_n)
            issue(i + 1, idx_n, rows_n, gsem_n)

        drain_gathers(rows_c, gsem_c)
        off = pl.multiple_of(base + i * SUPER, 8)
        pltpu.async_copy(rows_c, out_hbm.at[pl.ds(off, SUPER)], ssem_c)

    issue(0, idx0, rows0, gsem0)

    def body(i, carry):
        @pl.when(i % 2 == 0)
        def _():
            step(i, idx0, rows0, gsem0, ssem0, idx1, rows1, gsem1, ssem1)

        @pl.when(i % 2 == 1)
        def _():
            step(i, idx1, rows1, gsem1, ssem1, idx0, rows0, gsem0, ssem0)

        return carry

    lax.fori_loop(0, NSUP, body, 0)

    wait_store(rows1, ssem1)
    wait_store(rows0, ssem0)


def kernel(x, edge_index):
    src = edge_index[0]
    mesh = plsc.VectorSubcoreMesh(core_axis_name="c", subcore_axis_name="s")
    run = functools.partial(
        pl.kernel,
        out_type=jax.ShapeDtypeStruct((N_EDGES, D_FEAT), jnp.float32),
        mesh=mesh,
        scratch_types=[
            pltpu.VMEM((SUPER,), jnp.int32),
            pltpu.VMEM((SUPER,), jnp.int32),
            pltpu.VMEM((SUPER, D_FEAT), jnp.float32),
            pltpu.VMEM((SUPER, D_FEAT), jnp.float32),
            pltpu.SemaphoreType.DMA,
            pltpu.SemaphoreType.DMA,
            pltpu.SemaphoreType.DMA,
            pltpu.SemaphoreType.DMA,
        ],
    )(_gather_kernel)
    return run(x, src)


# speedup vs baseline: 5.7276x; 1.0012x over previous
import functools

import jax
import jax.numpy as jnp
from jax import lax
from jax.experimental import pallas as pl
from jax.experimental.pallas import tpu as pltpu
from jax.experimental.pallas import tpu_sc as plsc

N_NODES = 10000
N_EDGES = 320000
D_FEAT = 128

NUM_WORKERS = 32
E_PER_W = N_EDGES // NUM_WORKERS
SUPER = 400
G = 80
NG = SUPER // G
NSUP = E_PER_W // SUPER


def _gather_kernel(x_hbm, src_hbm, out_hbm,
                   idx_all, rows0, rows1,
                   gsem0, gsem1, ssem0, ssem1):
    cid = lax.axis_index("c")
    sid = lax.axis_index("s")
    wid = sid * 2 + cid
    base = wid * E_PER_W

    pltpu.sync_copy(src_hbm.at[pl.ds(pl.multiple_of(base, 8), E_PER_W)],
                    idx_all)

    def issue(i, rowsb, gsem):
        for j in range(NG):
            pltpu.async_copy(
                x_hbm.at[idx_all.at[pl.ds(i * SUPER + j * G, G)]],
                rowsb.at[pl.ds(j * G, G)],
                gsem,
            )

    def drain_gathers(rowsb, gsem):
        for j in range(NG):
            pltpu.make_async_copy(
                x_hbm.at[pl.ds(0, G)], rowsb.at[pl.ds(j * G, G)], gsem
            ).wait()

    def wait_store(rowsb, ssem):
        pltpu.make_async_copy(rowsb, out_hbm.at[pl.ds(0, SUPER)], ssem).wait()

    def step(i, rows_c, gsem_c, ssem_c, rows_n, gsem_n, ssem_n):
        @pl.when(i + 1 < NSUP)
        def _():
            @pl.when(i >= 1)
            def _():
                wait_store(rows_n, ssem_n)
            issue(i + 1, rows_n, gsem_n)

        drain_gathers(rows_c, gsem_c)
        off = pl.multiple_of(base + i * SUPER, 8)
        pltpu.async_copy(rows_c, out_hbm.at[pl.ds(off, SUPER)], ssem_c)

    issue(0, rows0, gsem0)

    def body(i, carry):
        @pl.when(i % 2 == 0)
        def _():
            step(i, rows0, gsem0, ssem0, rows1, gsem1, ssem1)

        @pl.when(i % 2 == 1)
        def _():
            step(i, rows1, gsem1, ssem1, rows0, gsem0, ssem0)

        return carry

    lax.fori_loop(0, NSUP, body, 0)

    wait_store(rows1, ssem1)
    wait_store(rows0, ssem0)


def kernel(x, edge_index):
    src = edge_index[0]
    mesh = plsc.VectorSubcoreMesh(core_axis_name="c", subcore_axis_name="s")
    run = functools.partial(
        pl.kernel,
        out_type=jax.ShapeDtypeStruct((N_EDGES, D_FEAT), jnp.float32),
        mesh=mesh,
        scratch_types=[
            pltpu.VMEM((E_PER_W,), jnp.int32),
            pltpu.VMEM((SUPER, D_FEAT), jnp.float32),
            pltpu.VMEM((SUPER, D_FEAT), jnp.float32),
            pltpu.SemaphoreType.DMA,
            pltpu.SemaphoreType.DMA,
            pltpu.SemaphoreType.DMA,
            pltpu.SemaphoreType.DMA,
        ],
    )(_gather_kernel)
    return run(x, src)


# speedup vs baseline: 6.1671x; 1.0767x over previous
import functools

import jax
import jax.numpy as jnp
from jax import lax
from jax.experimental import pallas as pl
from jax.experimental.pallas import tpu as pltpu
from jax.experimental.pallas import tpu_sc as plsc

N_NODES = 10000
N_EDGES = 320000
D_FEAT = 128

NUM_WORKERS = 32
E_PER_W = N_EDGES // NUM_WORKERS
SUPER = 400
G = 80
NG = SUPER // G
NSUP = E_PER_W // SUPER


def _gather_kernel(x_hbm, src_hbm, out_hbm,
                   idx_all, rows0, rows1,
                   gsem0, gsem1, ssem0, ssem1):
    cid = lax.axis_index("c")
    sid = lax.axis_index("s")
    wid = sid * 2 + cid
    base = wid * E_PER_W

    pltpu.sync_copy(src_hbm.at[pl.ds(pl.multiple_of(base, 8), E_PER_W)],
                    idx_all)

    def issue(i, rowsb, gsem):
        for j in range(NG):
            pltpu.async_copy(
                x_hbm.at[idx_all.at[pl.ds(i * SUPER + j * G, G)]],
                rowsb.at[pl.ds(j * G, G)],
                gsem,
            )

    def drain_gathers(rowsb, gsem):
        for j in range(NG):
            pltpu.make_async_copy(
                x_hbm.at[pl.ds(0, G)], rowsb.at[pl.ds(j * G, G)], gsem
            ).wait()

    def wait_store(rowsb, ssem):
        pltpu.make_async_copy(rowsb, out_hbm.at[pl.ds(0, SUPER)], ssem).wait()

    def step(i, rows_c, gsem_c, ssem_c, rows_n, gsem_n, ssem_n):
        @pl.when(i + 1 < NSUP)
        def _():
            @pl.when(i >= 1)
            def _():
                wait_store(rows_n, ssem_n)
            issue(i + 1, rows_n, gsem_n)

        drain_gathers(rows_c, gsem_c)
        off = pl.multiple_of(base + i * SUPER, 8)
        pltpu.async_copy(rows_c, out_hbm.at[pl.ds(off, SUPER)], ssem_c)

    issue(0, rows0, gsem0)

    def body(i, carry):
        @pl.when(i % 2 == 0)
        def _():
            step(i, rows0, gsem0, ssem0, rows1, gsem1, ssem1)

        @pl.when(i % 2 == 1)
        def _():
            step(i, rows1, gsem1, ssem1, rows0, gsem0, ssem0)

        return carry

    lax.fori_loop(0, NSUP, body, 0)

    wait_store(rows1, ssem1)
    wait_store(rows0, ssem0)


def kernel(x, edge_index):
    src = edge_index.reshape(2 * N_EDGES)
    mesh = plsc.VectorSubcoreMesh(core_axis_name="c", subcore_axis_name="s")
    run = functools.partial(
        pl.kernel,
        out_type=jax.ShapeDtypeStruct((N_EDGES, D_FEAT), jnp.float32),
        mesh=mesh,
        scratch_types=[
            pltpu.VMEM((E_PER_W,), jnp.int32),
            pltpu.VMEM((SUPER, D_FEAT), jnp.float32),
            pltpu.VMEM((SUPER, D_FEAT), jnp.float32),
            pltpu.SemaphoreType.DMA,
            pltpu.SemaphoreType.DMA,
            pltpu.SemaphoreType.DMA,
            pltpu.SemaphoreType.DMA,
        ],
    )(_gather_kernel)
    return run(x, src)
